# baseline (speedup 1.0000x reference)
import functools

import jax
import jax.numpy as jnp
from jax import lax
from jax.experimental import pallas as pl
from jax.experimental.pallas import tpu as pltpu
from jax.experimental.pallas import tpu_sc as plsc

ROWS = 128
COLS = 32768
DROP = 512
KEEP = COLS - DROP

LANES = 16
NUM_CORES = 2
NUM_SUBCORES = 16
NW = NUM_CORES * NUM_SUBCORES
SC_ROW0 = ROWS - NW
CHUNK = KEEP // 2
NACC = 4
STEP = NACC * LANES

_mesh = plsc.VectorSubcoreMesh(
    core_axis_name="c", subcore_axis_name="s",
    num_cores=NUM_CORES, num_subcores=NUM_SUBCORES,
)


@functools.partial(
    pl.kernel,
    out_type=jax.ShapeDtypeStruct((NW, LANES), jnp.float32),
    mesh=_mesh,
    scratch_types=[
        pltpu.VMEM((CHUNK,), jnp.float32),
        pltpu.VMEM((CHUNK,), jnp.float32),
        pltpu.VMEM((LANES,), jnp.float32),
        pltpu.SemaphoreType.DMA,
        pltpu.SemaphoreType.DMA,
    ],
)
def _row_means_sc(tok_hbm, out_hbm, buf0, buf1, res_v, sem0, sem1):
    wid = lax.axis_index("s") * NUM_CORES + lax.axis_index("c")
    row = SC_ROW0 + wid
    bufs = (buf0, buf1)
    sems = (sem0, sem1)

    copies = [None, None]
    copies[0] = pltpu.async_copy(
        tok_hbm.at[row, pl.ds(DROP, CHUNK)], bufs[0], sems[0])
    copies[1] = pltpu.async_copy(
        tok_hbm.at[row, pl.ds(DROP + CHUNK, CHUNK)], bufs[1], sems[1])

    zero = jnp.zeros((LANES,), jnp.float32)
    total = zero
    for c in range(2):
        copies[c].wait()
        buf = bufs[c]

        @plsc.parallel_loop(0, CHUNK, STEP, carry=(zero,) * NACC)
        def accs(i, a):
            return tuple(
                a[u] + buf[pl.ds(i + u * LANES, LANES)] for u in range(NACC)
            )

        for u in range(NACC):
            total = total + accs[u]

    lane_ids = lax.iota(jnp.int32, LANES)
    for k in (1, 2, 4, 8):
        total = total + jnp.take(total, lane_ids ^ k)
    res_v[...] = total * (1.0 / KEEP)
    pltpu.sync_copy(res_v, out_hbm.at[wid])


NS = 2
RB = 24
PART = SC_ROW0 // NS
NSTEP = PART // RB


def _tc_body(*refs):
    ins = refs[:NS]
    outs = refs[NS:]
    cols = lax.broadcasted_iota(jnp.int32, (RB, COLS), 1)
    m = cols >= DROP
    for a, o in zip(ins, outs):
        x = jnp.where(m, a[...], 0.0)
        o[...] = jnp.sum(x, axis=1, keepdims=True) * (1.0 / KEEP)


def _mk_in(s):
    return pl.BlockSpec((RB, COLS), lambda i, s=s: (i + s * NSTEP, 0))


def _mk_out(s):
    return pl.BlockSpec((RB, 1), lambda i, s=s: (i + s * NSTEP, 0))


_tc_means = pl.pallas_call(
    _tc_body,
    grid=(NSTEP,),
    in_specs=[_mk_in(s) for s in range(NS)],
    out_specs=[_mk_out(s) for s in range(NS)],
    out_shape=[jax.ShapeDtypeStruct((SC_ROW0, 1), jnp.float32)] * NS,
    compiler_params=pltpu.CompilerParams(
        dimension_semantics=("parallel",),
    ),
)


def kernel(tokens):
    sc_block = _row_means_sc(tokens)
    tc_parts = _tc_means(*([tokens] * NS))
    rid = lax.broadcasted_iota(jnp.int32, (SC_ROW0, 1), 0)
    tc = tc_parts[0]
    for s in range(1, NS):
        tc = jnp.where(rid < s * PART, tc, tc_parts[s])
    return jnp.concatenate([tc, sc_block[:, :1]], axis=0)


# speedup vs baseline: 2.8788x; 2.8788x over previous
import jax
import jax.numpy as jnp
from jax import lax
from jax.experimental import pallas as pl
from jax.experimental.pallas import tpu as pltpu

ROWS = 128
COLS = 32768
DROP = 512
KEEP = COLS - DROP
RB = 8
NCHUNK = ROWS // RB


def _tc_body(tok_hbm, out_ref, bufs, sems):
    copies = []
    for c in range(NCHUNK):
        cp = pltpu.make_async_copy(
            tok_hbm.at[pl.ds(c * RB, RB), pl.ds(DROP, KEEP)],
            bufs.at[c], sems.at[c])
        cp.start()
        copies.append(cp)
    for c in range(NCHUNK):
        copies[c].wait()
        x = bufs[c]
        out_ref[pl.ds(c * RB, RB), :] = (
            jnp.sum(x, axis=1, keepdims=True) * (1.0 / KEEP))


_tc_mean = pl.pallas_call(
    _tc_body,
    in_specs=[pl.BlockSpec(memory_space=pl.ANY)],
    out_specs=pl.BlockSpec(memory_space=pltpu.MemorySpace.VMEM),
    out_shape=jax.ShapeDtypeStruct((ROWS, 1), jnp.float32),
    scratch_shapes=[
        pltpu.VMEM((NCHUNK, RB, KEEP), jnp.float32),
        pltpu.SemaphoreType.DMA((NCHUNK,)),
    ],
)


def kernel(tokens):
    return _tc_mean(tokens)


# speedup vs baseline: 3.2312x; 1.1224x over previous
import jax
import jax.numpy as jnp
from jax import lax
from jax.experimental import pallas as pl
from jax.experimental.pallas import tpu as pltpu

ROWS = 128
COLS = 32768
DROP = 512
KEEP = COLS - DROP
RB = 16
NCHUNK = ROWS // RB


def _tc_body(tok_hbm, out_ref, bufs, sems):
    copies = []
    for c in range(NCHUNK):
        cp = pltpu.make_async_copy(
            tok_hbm.at[pl.ds(c * RB, RB), :], bufs.at[c], sems.at[c])
        cp.start()
        copies.append(cp)
    cols = lax.broadcasted_iota(jnp.int32, (RB, COLS), 1)
    m = cols >= DROP
    for c in range(NCHUNK):
        copies[c].wait()
        x = jnp.where(m, bufs[c], 0.0)
        out_ref[pl.ds(c * RB, RB), :] = (
            jnp.sum(x, axis=1, keepdims=True) * (1.0 / KEEP))


_tc_mean = pl.pallas_call(
    _tc_body,
    in_specs=[pl.BlockSpec(memory_space=pl.ANY)],
    out_specs=pl.BlockSpec(memory_space=pltpu.MemorySpace.VMEM),
    out_shape=jax.ShapeDtypeStruct((ROWS, 1), jnp.float32),
    scratch_shapes=[
        pltpu.VMEM((NCHUNK, RB, COLS), jnp.float32),
        pltpu.SemaphoreType.DMA((NCHUNK,)),
    ],
)


def kernel(tokens):
    return _tc_mean(tokens)
